# Initial kernel scaffold; baseline (speedup 1.0000x reference)
import functools

import jax
import jax.numpy as jnp
from jax import lax
from jax.experimental import pallas as pl
from jax.experimental.pallas import tpu as pltpu
from jax.experimental.pallas import tpu_sc as plsc

NN = 10000
F = 128
E = 160000
NR = 3
NC = 2
NS = 16
NW = NC * NS
EPW = E // NW
CHUNK = 125
NCHUNK = EPW // CHUNK
RPT = NN // NS
ZB = RPT // 5


def _make_agg_kernel(with_deg):
  mesh = plsc.VectorSubcoreMesh(core_axis_name="c", subcore_axis_name="s")
  out_type = [jax.ShapeDtypeStruct((NR, NC, NN, F), jnp.float32)]
  if with_deg:
    out_type.append(jax.ShapeDtypeStruct((NR, NC, NN, 16), jnp.float32))
  scratch = [
      pltpu.VMEM_SHARED((NN, F), jnp.float32),
      pltpu.VMEM((NCHUNK, CHUNK), jnp.int32),
      pltpu.VMEM((NCHUNK, CHUNK), jnp.int32),
      pltpu.VMEM((CHUNK, F), jnp.float32),
      pltpu.VMEM((ZB, F), jnp.float32),
      pltpu.SemaphoreType.DMA,
  ]
  if with_deg:
    scratch += [
        pltpu.VMEM_SHARED((NN, 16), jnp.float32),
        pltpu.VMEM((CHUNK, 16), jnp.float32),
        pltpu.VMEM((ZB, 16), jnp.float32),
    ]

  @functools.partial(pl.kernel, mesh=mesh, out_type=out_type,
                     scratch_types=scratch)
  def k(h_hbm, src_hbm, dst_hbm, agg_out, *rest):
    if with_deg:
      (deg_out, agg_sh, src_v, dst_v, rows0, zbuf, sem0,
       deg_sh, ones_v, z16) = rest
    else:
      agg_sh, src_v, dst_v, rows0, zbuf, sem0 = rest
    cid = lax.axis_index("c")
    sid = lax.axis_index("s")
    wid = cid * NS + sid
    row0 = sid * RPT

    def zrow(i, _):
      for cc in range(F // 16):
        zbuf[i, pl.ds(cc * 16, 16)] = jnp.zeros((16,), jnp.float32)
      if with_deg:
        ones_v[i, pl.ds(0, 16)] = jnp.ones((16,), jnp.float32)
        z16[i, pl.ds(0, 16)] = jnp.zeros((16,), jnp.float32)
      return 0
    lax.fori_loop(0, ZB, zrow, 0)

    for r in range(NR):
      for t in range(RPT // ZB):
        pltpu.sync_copy(zbuf, agg_sh.at[pl.ds(row0 + t * ZB, ZB)])
        if with_deg:
          pltpu.sync_copy(z16, deg_sh.at[pl.ds(row0 + t * ZB, ZB)])
      plsc.subcore_barrier()

      pltpu.sync_copy(src_hbm.at[r, wid], src_v)
      pltpu.sync_copy(dst_hbm.at[r, wid], dst_v)

      def chunk(j, _):
        pltpu.async_copy(h_hbm.at[src_v.at[j]], rows0, sem0).wait()
        pltpu.sync_copy(rows0, agg_sh.at[dst_v.at[j]], add=True)
        if with_deg:
          pltpu.sync_copy(ones_v, deg_sh.at[dst_v.at[j]], add=True)
        return 0
      lax.fori_loop(0, NCHUNK, chunk, 0)
      plsc.subcore_barrier()

      pltpu.sync_copy(agg_sh.at[pl.ds(row0, RPT)],
                      agg_out.at[r, cid, pl.ds(row0, RPT)])
      if with_deg:
        pltpu.sync_copy(deg_sh.at[pl.ds(row0, RPT)],
                        deg_out.at[r, cid, pl.ds(row0, RPT)])
      plsc.subcore_barrier()

  return k


_agg_deg = _make_agg_kernel(True)
_agg_only = _make_agg_kernel(False)

BLK = 2000


def _dense_body(relu, h_ref, agg_ref, deg_ref, ws_ref, wn_ref, b_ref, out_ref):
  ws = ws_ref[0] + ws_ref[1] + ws_ref[2]
  acc = jnp.dot(h_ref[...], ws, preferred_element_type=jnp.float32)
  for r in range(NR):
    agg = agg_ref[r, 0] + agg_ref[r, 1]
    deg = deg_ref[r, 0, :, :1] + deg_ref[r, 1, :, :1]
    mean = agg * (1.0 / jnp.maximum(deg, 1.0))
    acc = acc + jnp.dot(mean, wn_ref[r], preferred_element_type=jnp.float32)
  acc = acc + (b_ref[0] + b_ref[1] + b_ref[2])[None, :]
  if relu:
    acc = jnp.maximum(acc, 0.0)
  out_ref[...] = acc


def _dense_layer(relu, h, agg, deg, w_self, w_neigh, b):
  grid = (NN // BLK,)
  return pl.pallas_call(
      functools.partial(_dense_body, relu),
      grid=grid,
      in_specs=[
          pl.BlockSpec((BLK, F), lambda i: (i, 0)),
          pl.BlockSpec((NR, NC, BLK, F), lambda i: (0, 0, i, 0)),
          pl.BlockSpec((NR, NC, BLK, 16), lambda i: (0, 0, i, 0)),
          pl.BlockSpec((NR, F, F), lambda i: (0, 0, 0)),
          pl.BlockSpec((NR, F, F), lambda i: (0, 0, 0)),
          pl.BlockSpec((NR, F), lambda i: (0, 0)),
      ],
      out_specs=pl.BlockSpec((BLK, F), lambda i: (i, 0)),
      out_shape=jax.ShapeDtypeStruct((NN, F), jnp.float32),
  )(h, agg, deg, w_self, w_neigh, b)


@jax.jit
def kernel(x, edge_index_follows, edge_index_likes, edge_index_views,
           W_self1, W_neigh1, b1, W_self2, W_neigh2, b2):
  eis = [edge_index_follows, edge_index_likes, edge_index_views]
  src = jnp.stack([e[0] for e in eis]).astype(jnp.int32).reshape(
      NR, NW, NCHUNK, CHUNK)
  dst = jnp.stack([e[1] for e in eis]).astype(jnp.int32).reshape(
      NR, NW, NCHUNK, CHUNK)

  agg1, deg = _agg_deg(x, src, dst)
  h1 = _dense_layer(True, x, agg1, deg, W_self1, W_neigh1, b1)
  agg2 = _agg_only(h1, src, dst)
  return _dense_layer(False, h1, agg2, deg, W_self2, W_neigh2, b2)


# speedup vs baseline: 6.8091x; 6.8091x over previous
import functools

import jax
import jax.numpy as jnp
from jax import lax
from jax.experimental import pallas as pl
from jax.experimental.pallas import tpu as pltpu
from jax.experimental.pallas import tpu_sc as plsc

NN = 10000
F = 128
E = 160000
NR = 3
NC = 2
NS = 16
NW = NC * NS
EPW = E // NW
CHUNK = 125
NCHUNK = EPW // CHUNK
NNP = 10240
RPT = NNP // NS
ZB = RPT // 5

_MESH = plsc.VectorSubcoreMesh(core_axis_name="c", subcore_axis_name="s")


@functools.partial(
    pl.kernel, mesh=_MESH,
    out_type=[jax.ShapeDtypeStruct((NR, NC, NNP, F), jnp.float32)],
    scratch_types=[
        pltpu.VMEM_SHARED((NNP, F), jnp.float32),
        pltpu.VMEM((NCHUNK, CHUNK), jnp.int32),
        pltpu.VMEM((NCHUNK, CHUNK), jnp.int32),
        pltpu.VMEM((CHUNK, F), jnp.float32),
        pltpu.VMEM((ZB, F), jnp.float32),
        pltpu.SemaphoreType.DMA,
    ])
def _agg_kernel(h_hbm, src_hbm, dst_hbm, agg_out,
                agg_sh, src_v, dst_v, rows0, zbuf, sem0):
  cid = lax.axis_index("c")
  sid = lax.axis_index("s")
  wid = cid * NS + sid
  row0 = sid * RPT

  def zrow(i, _):
    for cc in range(F // 16):
      zbuf[i, pl.ds(cc * 16, 16)] = jnp.zeros((16,), jnp.float32)
    return 0
  lax.fori_loop(0, ZB, zrow, 0)

  for r in range(NR):
    for t in range(RPT // ZB):
      pltpu.sync_copy(zbuf, agg_sh.at[pl.ds(row0 + t * ZB, ZB)])
    plsc.subcore_barrier()

    pltpu.sync_copy(src_hbm.at[r, wid], src_v)
    pltpu.sync_copy(dst_hbm.at[r, wid], dst_v)

    def chunk(j, _):
      pltpu.async_copy(h_hbm.at[src_v.at[j]], rows0, sem0).wait()
      pltpu.sync_copy(rows0, agg_sh.at[dst_v.at[j]], add=True)
      return 0
    lax.fori_loop(0, NCHUNK, chunk, 0)
    plsc.subcore_barrier()

    pltpu.sync_copy(agg_sh.at[pl.ds(row0, RPT)],
                    agg_out.at[r, cid, pl.ds(row0, RPT)])
    plsc.subcore_barrier()


@functools.partial(
    pl.kernel, mesh=_MESH,
    out_type=[jax.ShapeDtypeStruct((NR, NW, NNP), jnp.float32)],
    scratch_types=[
        pltpu.VMEM((NNP,), jnp.float32),
        pltpu.VMEM((EPW + 16,), jnp.int32),
    ],
    compiler_params=pltpu.CompilerParams(needs_layout_passes=False,
                                         use_tc_tiling_on_sc=False))
def _deg_kernel(dstf_hbm, deg_out, deg_local, dst_loc):
  cid = lax.axis_index("c")
  sid = lax.axis_index("s")
  wid = cid * NS + sid
  ones = jnp.ones((16,), jnp.float32)
  nvec = (EPW + 15) // 16

  dst_loc[pl.ds(EPW - 16 + 16, 16)] = jnp.full((16,), NN, jnp.int32)

  for r in range(NR):
    def z(i, _):
      deg_local[pl.ds(i * 16, 16)] = jnp.zeros((16,), jnp.float32)
      return 0
    lax.fori_loop(0, NNP // 16, z, 0)

    pltpu.sync_copy(dstf_hbm.at[r, wid], dst_loc.at[pl.ds(0, EPW)])

    def step(i, _):
      idx = dst_loc[pl.ds(i * 16, 16)]
      plsc.addupdate_scatter(deg_local, [idx], ones)
      return 0
    lax.fori_loop(0, nvec, step, 0)

    pltpu.sync_copy(deg_local, deg_out.at[r, wid])


BLK = 2048


def _dense_body(relu, h_ref, agg_ref, deg_ref, ws_ref, wn_ref, b_ref, out_ref):
  ws = ws_ref[0] + ws_ref[1] + ws_ref[2]
  acc = jnp.dot(h_ref[...], ws, preferred_element_type=jnp.float32)
  for r in range(NR):
    agg = agg_ref[r, 0] + agg_ref[r, 1]
    deg = jnp.sum(deg_ref[r], axis=0)
    mean = agg * (1.0 / jnp.maximum(deg, 1.0))[:, None]
    acc = acc + jnp.dot(mean, wn_ref[r], preferred_element_type=jnp.float32)
  acc = acc + (b_ref[0] + b_ref[1] + b_ref[2])[None, :]
  if relu:
    acc = jnp.maximum(acc, 0.0)
  out_ref[...] = acc


def _dense_layer(relu, h, agg, deg, w_self, w_neigh, b):
  grid = (NNP // BLK,)
  return pl.pallas_call(
      functools.partial(_dense_body, relu),
      grid=grid,
      in_specs=[
          pl.BlockSpec((BLK, F), lambda i: (i, 0)),
          pl.BlockSpec((NR, NC, BLK, F), lambda i: (0, 0, i, 0)),
          pl.BlockSpec((NR, NW, BLK), lambda i: (0, 0, i)),
          pl.BlockSpec((NR, F, F), lambda i: (0, 0, 0)),
          pl.BlockSpec((NR, F, F), lambda i: (0, 0, 0)),
          pl.BlockSpec((NR, F), lambda i: (0, 0)),
      ],
      out_specs=pl.BlockSpec((BLK, F), lambda i: (i, 0)),
      out_shape=jax.ShapeDtypeStruct((NNP, F), jnp.float32),
  )(h, agg, deg, w_self, w_neigh, b)


@jax.jit
def kernel(x, edge_index_follows, edge_index_likes, edge_index_views,
           W_self1, W_neigh1, b1, W_self2, W_neigh2, b2):
  eis = [edge_index_follows, edge_index_likes, edge_index_views]
  src = jnp.stack([e[0] for e in eis]).astype(jnp.int32).reshape(
      NR, NW, NCHUNK, CHUNK)
  dst = jnp.stack([e[1] for e in eis]).astype(jnp.int32).reshape(
      NR, NW, NCHUNK, CHUNK)
  dstf = dst.reshape(NR, NW, EPW)
  x_p = jnp.pad(x, ((0, NNP - NN), (0, 0)))

  (deg,) = _deg_kernel(dstf)
  (agg1,) = _agg_kernel(x_p, src, dst)
  h1 = _dense_layer(True, x_p, agg1, deg, W_self1, W_neigh1, b1)
  (agg2,) = _agg_kernel(h1, src, dst)
  out = _dense_layer(False, h1, agg2, deg, W_self2, W_neigh2, b2)
  return out[:NN]


# speedup vs baseline: 9.9677x; 1.4639x over previous
import functools

import jax
import jax.numpy as jnp
from jax import lax
from jax.experimental import pallas as pl
from jax.experimental.pallas import tpu as pltpu
from jax.experimental.pallas import tpu_sc as plsc

NN = 10000
F = 128
E = 160000
NR = 3
NC = 2
NS = 16
NW = NC * NS
EPW = E // NW
CHUNK = 125
NCHUNK = EPW // CHUNK
NNP = 10240
RPT = NNP // NS
ZB = RPT // 5

_MESH = plsc.VectorSubcoreMesh(core_axis_name="c", subcore_axis_name="s")


@functools.partial(
    pl.kernel, mesh=_MESH,
    out_type=[jax.ShapeDtypeStruct((NR, NC, NNP, F), jnp.float32)],
    scratch_types=[
        pltpu.VMEM_SHARED((NNP, F), jnp.float32),
        pltpu.VMEM((NCHUNK, CHUNK), jnp.int32),
        pltpu.VMEM((NCHUNK, CHUNK), jnp.int32),
        pltpu.VMEM((ZB, F), jnp.float32),
        pltpu.VMEM((ZB, F), jnp.float32),
        pltpu.SemaphoreType.DMA,
        pltpu.SemaphoreType.DMA,
    ])
def _agg_kernel(h_hbm, src_hbm, dst_hbm, agg_out,
                agg_sh, src_v, dst_v, rows0, rows1, sem0, sem1):
  cid = lax.axis_index("c")
  sid = lax.axis_index("s")
  wid = cid * NS + sid
  row0 = sid * RPT

  for r in range(NR):
    def zrow(i, _):
      for cc in range(F // 16):
        rows1[i, pl.ds(cc * 16, 16)] = jnp.zeros((16,), jnp.float32)
      return 0
    lax.fori_loop(0, ZB, zrow, 0)
    for t in range(RPT // ZB):
      pltpu.sync_copy(rows1, agg_sh.at[pl.ds(row0 + t * ZB, ZB)])
    plsc.subcore_barrier()

    pltpu.sync_copy(src_hbm.at[r, wid], src_v)
    pltpu.sync_copy(dst_hbm.at[r, wid], dst_v)

    r0 = rows0.at[pl.ds(0, CHUNK)]
    r1 = rows1.at[pl.ds(0, CHUNK)]
    pltpu.async_copy(h_hbm.at[src_v.at[0]], r0, sem0)

    def chunk2(jj, _):
      j = jj * 2
      pltpu.async_copy(h_hbm.at[src_v.at[j + 1]], r1, sem1)
      pltpu.make_async_copy(h_hbm.at[src_v.at[j]], r0, sem0).wait()
      pltpu.sync_copy(r0, agg_sh.at[dst_v.at[j]], add=True)

      @pl.when(j + 2 < NCHUNK)
      def _():
        pltpu.async_copy(h_hbm.at[src_v.at[j + 2]], r0, sem0)
      pltpu.make_async_copy(h_hbm.at[src_v.at[j + 1]], r1, sem1).wait()
      pltpu.sync_copy(r1, agg_sh.at[dst_v.at[j + 1]], add=True)
      return 0
    lax.fori_loop(0, NCHUNK // 2, chunk2, 0)
    plsc.subcore_barrier()

    pltpu.sync_copy(agg_sh.at[pl.ds(row0, RPT)],
                    agg_out.at[r, cid, pl.ds(row0, RPT)])
    plsc.subcore_barrier()


@functools.partial(
    pl.kernel, mesh=_MESH,
    out_type=[jax.ShapeDtypeStruct((NR, NW, NNP), jnp.float32)],
    scratch_types=[
        pltpu.VMEM((NNP,), jnp.float32),
        pltpu.VMEM((EPW + 16,), jnp.int32),
    ],
    compiler_params=pltpu.CompilerParams(needs_layout_passes=False,
                                         use_tc_tiling_on_sc=False))
def _deg_kernel(dstf_hbm, deg_out, deg_local, dst_loc):
  cid = lax.axis_index("c")
  sid = lax.axis_index("s")
  wid = cid * NS + sid
  ones = jnp.ones((16,), jnp.float32)
  nvec = (EPW + 15) // 16

  dst_loc[pl.ds(EPW - 16 + 16, 16)] = jnp.full((16,), NN, jnp.int32)

  for r in range(NR):
    def z(i, _):
      deg_local[pl.ds(i * 16, 16)] = jnp.zeros((16,), jnp.float32)
      return 0
    lax.fori_loop(0, NNP // 16, z, 0)

    pltpu.sync_copy(dstf_hbm.at[r, wid], dst_loc.at[pl.ds(0, EPW)])

    def step(i, _):
      idx = dst_loc[pl.ds(i * 16, 16)]
      plsc.addupdate_scatter(deg_local, [idx], ones)
      return 0
    lax.fori_loop(0, nvec, step, 0)

    pltpu.sync_copy(deg_local, deg_out.at[r, wid])


BLK = 2048


def _dense_body(relu, h_ref, agg_ref, deg_ref, ws_ref, wn_ref, b_ref, out_ref):
  ws = ws_ref[0] + ws_ref[1] + ws_ref[2]
  acc = jnp.dot(h_ref[...], ws, preferred_element_type=jnp.float32)
  for r in range(NR):
    agg = agg_ref[r, 0] + agg_ref[r, 1]
    deg = jnp.sum(deg_ref[r], axis=0)
    mean = agg * (1.0 / jnp.maximum(deg, 1.0))[:, None]
    acc = acc + jnp.dot(mean, wn_ref[r], preferred_element_type=jnp.float32)
  acc = acc + (b_ref[0] + b_ref[1] + b_ref[2])[None, :]
  if relu:
    acc = jnp.maximum(acc, 0.0)
  out_ref[...] = acc


def _dense_layer(relu, h, agg, deg, w_self, w_neigh, b):
  grid = (NNP // BLK,)
  return pl.pallas_call(
      functools.partial(_dense_body, relu),
      grid=grid,
      in_specs=[
          pl.BlockSpec((BLK, F), lambda i: (i, 0)),
          pl.BlockSpec((NR, NC, BLK, F), lambda i: (0, 0, i, 0)),
          pl.BlockSpec((NR, NW, BLK), lambda i: (0, 0, i)),
          pl.BlockSpec((NR, F, F), lambda i: (0, 0, 0)),
          pl.BlockSpec((NR, F, F), lambda i: (0, 0, 0)),
          pl.BlockSpec((NR, F), lambda i: (0, 0)),
      ],
      out_specs=pl.BlockSpec((BLK, F), lambda i: (i, 0)),
      out_shape=jax.ShapeDtypeStruct((NNP, F), jnp.float32),
  )(h, agg, deg, w_self, w_neigh, b)


@jax.jit
def kernel(x, edge_index_follows, edge_index_likes, edge_index_views,
           W_self1, W_neigh1, b1, W_self2, W_neigh2, b2):
  eis = [edge_index_follows, edge_index_likes, edge_index_views]
  src = jnp.stack([e[0] for e in eis]).astype(jnp.int32).reshape(
      NR, NW, NCHUNK, CHUNK)
  dst = jnp.stack([e[1] for e in eis]).astype(jnp.int32).reshape(
      NR, NW, NCHUNK, CHUNK)
  dstf = dst.reshape(NR, NW, EPW)
  x_p = jnp.pad(x, ((0, NNP - NN), (0, 0)))

  (deg,) = _deg_kernel(dstf)
  (agg1,) = _agg_kernel(x_p, src, dst)
  h1 = _dense_layer(True, x_p, agg1, deg, W_self1, W_neigh1, b1)
  (agg2,) = _agg_kernel(h1, src, dst)
  out = _dense_layer(False, h1, agg2, deg, W_self2, W_neigh2, b2)
  return out[:NN]
